# Initial kernel scaffold; baseline (speedup 1.0000x reference)
import functools

import jax
import jax.numpy as jnp
from jax import lax
from jax.experimental import pallas as pl
from jax.experimental.pallas import tpu as pltpu
from jax.experimental.pallas import tpu_sc as plsc

N = 10000
E = 320000
D = 128
R = 8

NC = 2
NS = 16
L = 16
NW = NC * NS

NP = 10240
EP = 327680
EPT = EP // NW
MC = 1024
SUB = 128
CW = NP // NS

_f32 = jnp.float32


def _tc0_body(f_ref, w2_ref, wcat_ref, zh_ref, a_ref):
    zh = jnp.dot(f_ref[...], w2_ref[...], preferred_element_type=_f32)
    zh_ref[...] = zh
    a_ref[...] = jnp.dot(zh, wcat_ref[...], preferred_element_type=_f32)


def _tc0(feature, w2, wcat):
    return pl.pallas_call(
        _tc0_body,
        grid=(25,),
        in_specs=[
            pl.BlockSpec((400, D), lambda i: (i, 0)),
            pl.BlockSpec((D, R * D), lambda i: (0, 0)),
            pl.BlockSpec((R * D, 2 * R), lambda i: (0, 0)),
        ],
        out_specs=[
            pl.BlockSpec((400, R * D), lambda i: (i, 0)),
            pl.BlockSpec((400, 2 * R), lambda i: (i, 0)),
        ],
        out_shape=[
            jax.ShapeDtypeStruct((N, R * D), _f32),
            jax.ShapeDtypeStruct((N, 2 * R), _f32),
        ],
    )(feature, w2, wcat)


def _sc_a_body(is_hbm, id_hbm, as_hbm, ad_hbm,
               e_hbm, mpart_hbm,
               as_t, m_t, is_buf, id_buf, ad_buf, e_buf,
               red0, red1, stage, sem):
    cid = lax.axis_index("c")
    sid = lax.axis_index("s")
    wid = cid * NS + sid
    base_row = wid * (EPT // 128)

    pltpu.sync_copy(as_hbm, as_t)

    @pl.loop(0, NP // L)
    def _zero(v):
        m_t[pl.ds(v * L, L)] = jnp.zeros((L,), _f32)

    @pl.loop(0, EPT // MC)
    def _chunk(g):
        row = base_row + g * (MC // 128)
        pltpu.sync_copy(is_hbm.at[pl.ds(row, MC // 128)], is_buf)
        pltpu.sync_copy(id_hbm.at[pl.ds(row, MC // 128)], id_buf)

        @pl.loop(0, MC // 128)
        def _sub(j):
            pltpu.async_copy(ad_hbm.at[id_buf.at[j]], ad_buf.at[j], sem).wait()

            @pl.loop(0, 128 // L)
            def _vec(k):
                sl = pl.ds(k * L, L)
                is16 = is_buf[j, sl]
                id16 = id_buf[j, sl]
                as16 = plsc.load_gather(as_t, [is16])
                e16 = jnp.maximum(as16 + ad_buf[j, sl], 0.0)
                e_buf[j, sl] = e16
                d16 = lax.shift_right_logical(id16, 3)
                cur = plsc.load_gather(m_t, [d16])
                upd = jnp.maximum(cur, e16)
                plsc.store_scatter(m_t, [d16], upd)
                chk = plsc.load_gather(m_t, [d16])

                @pl.when(jnp.any(chk < upd))
                def _dup_fallback():
                    for t in range(L):
                        dt = lax.shift_right_logical(id_buf[j, k * L + t], 3)
                        ev = e_buf[j, k * L + t]
                        m_t[dt] = jnp.maximum(m_t[dt], ev)

        pltpu.sync_copy(e_buf, e_hbm.at[pl.ds(row, MC // 128)])

    pltpu.sync_copy(m_t, stage.at[sid])
    plsc.subcore_barrier()
    pltpu.sync_copy(stage.at[0, pl.ds(sid * CW, CW)], red0)
    for r in range(1, NS):
        pltpu.sync_copy(stage.at[r, pl.ds(sid * CW, CW)], red1)

        @pl.loop(0, CW // L)
        def _red(v):
            sl = pl.ds(v * L, L)
            red0[sl] = jnp.maximum(red0[sl], red1[sl])

    pltpu.sync_copy(red0, mpart_hbm.at[cid, pl.ds(sid * CW, CW)])


def _sc_a(is2d, id2d, a_s_flat, a_d_flat):
    mesh = plsc.VectorSubcoreMesh(core_axis_name="c", subcore_axis_name="s",
                                  num_cores=NC, num_subcores=NS)
    kern = pl.kernel(
        _sc_a_body,
        out_type=[
            jax.ShapeDtypeStruct((EP // 128, 128), _f32),
            jax.ShapeDtypeStruct((NC, NP), _f32),
        ],
        mesh=mesh,
        scratch_types=[
            pltpu.VMEM((N * R,), _f32),
            pltpu.VMEM((NP,), _f32),
            pltpu.VMEM((MC // 128, 128), jnp.int32),
            pltpu.VMEM((MC // 128, 128), jnp.int32),
            pltpu.VMEM((MC // 128, 128), _f32),
            pltpu.VMEM((MC // 128, 128), _f32),
            pltpu.VMEM((CW,), _f32),
            pltpu.VMEM((CW,), _f32),
            pltpu.VMEM_SHARED((NS, NP), _f32),
            pltpu.SemaphoreType.DMA,
        ],
    )
    return kern(is2d, id2d, a_s_flat, a_d_flat)


def _sc_b_body(is_hbm, id_hbm, e_hbm, mpart_hbm, zh_hbm,
               hbar_hbm, den_hbm,
               m_t, mtmp, is_buf, id_buf, dst_buf, e_buf, ex_buf,
               rows, zbuf, zden, sph, spd, sem):
    cid = lax.axis_index("c")
    sid = lax.axis_index("s")
    wid = cid * NS + sid
    base_row = wid * (EPT // 128)

    pltpu.sync_copy(mpart_hbm.at[0], m_t)
    pltpu.sync_copy(mpart_hbm.at[1], mtmp)

    @pl.loop(0, NP // L)
    def _mmax(v):
        sl = pl.ds(v * L, L)
        m_t[sl] = jnp.maximum(m_t[sl], mtmp[sl])

    @pl.loop(0, L)
    def _zb(i):
        @pl.loop(0, D // L)
        def _zbi(k):
            zbuf[i, pl.ds(k * L, L)] = jnp.zeros((L,), _f32)

    @pl.loop(0, CW // L)
    def _zd(v):
        zden[pl.ds(v * L, L)] = jnp.zeros((L,), _f32)

    @pl.loop(0, CW // L)
    def _zh_rows(c):
        pltpu.sync_copy(zbuf, sph.at[pl.ds(sid * CW + c * L, L)])

    pltpu.sync_copy(zden, spd.at[pl.ds(sid * CW, CW)])
    plsc.subcore_barrier()

    @pl.loop(0, EPT // MC)
    def _chunk(g):
        row = base_row + g * (MC // 128)
        pltpu.sync_copy(is_hbm.at[pl.ds(row, MC // 128)], is_buf)
        pltpu.sync_copy(id_hbm.at[pl.ds(row, MC // 128)], id_buf)
        pltpu.sync_copy(e_hbm.at[pl.ds(row, MC // 128)], e_buf)

        @pl.loop(0, MC // SUB)
        def _sub(j):
            @pl.loop(0, SUB // L)
            def _vec(k):
                sl = pl.ds(k * L, L)
                d16 = lax.shift_right_logical(id_buf[j, sl], 3)
                dst_buf[j, sl] = d16
                mv = plsc.load_gather(m_t, [d16])
                ex_buf[j, sl] = jnp.exp(e_buf[j, sl] - mv)

            pltpu.sync_copy(ex_buf.at[j], spd.at[dst_buf.at[j]], add=True)
            pltpu.async_copy(zh_hbm.at[is_buf.at[j]], rows, sem).wait()

            @pl.loop(0, SUB)
            def _scale(r):
                exv = jnp.full((L,), ex_buf[j, r], _f32)

                @pl.loop(0, D // L)
                def _sc(c):
                    sl = pl.ds(c * L, L)
                    rows[r, sl] = rows[r, sl] * exv

            pltpu.sync_copy(rows, sph.at[dst_buf.at[j]], add=True)

    plsc.subcore_barrier()
    pltpu.sync_copy(sph.at[pl.ds(sid * CW, CW)],
                    hbar_hbm.at[cid, pl.ds(sid * CW, CW)])
    pltpu.sync_copy(spd.at[pl.ds(sid * CW, CW)],
                    den_hbm.at[cid, pl.ds(sid * CW, CW)])


def _sc_b(is2d, id2d, e2d, m_part, zh_t):
    mesh = plsc.VectorSubcoreMesh(core_axis_name="c", subcore_axis_name="s",
                                  num_cores=NC, num_subcores=NS)
    kern = pl.kernel(
        _sc_b_body,
        out_type=[
            jax.ShapeDtypeStruct((NC, NP, D), _f32),
            jax.ShapeDtypeStruct((NC, NP), _f32),
        ],
        mesh=mesh,
        scratch_types=[
            pltpu.VMEM((NP,), _f32),
            pltpu.VMEM((NP,), _f32),
            pltpu.VMEM((MC // 128, 128), jnp.int32),
            pltpu.VMEM((MC // 128, 128), jnp.int32),
            pltpu.VMEM((MC // 128, 128), jnp.int32),
            pltpu.VMEM((MC // 128, 128), _f32),
            pltpu.VMEM((MC // 128, 128), _f32),
            pltpu.VMEM((SUB, D), _f32),
            pltpu.VMEM((L, D), _f32),
            pltpu.VMEM((CW,), _f32),
            pltpu.VMEM_SHARED((NP, D), _f32),
            pltpu.VMEM_SHARED((NP,), _f32),
            pltpu.SemaphoreType.DMA,
        ],
    )
    return kern(is2d, id2d, e2d, m_part, zh_t)


def _tcc_body(h0_ref, h1_ref, d0_ref, d1_ref, out_ref):
    den = d0_ref[...] + d1_ref[...]
    den = jnp.where(den == 0.0, 1.0, den)
    out_ref[...] = (h0_ref[...] + h1_ref[...]) / den


def _tcc(h0, h1, d0, d1):
    return pl.pallas_call(
        _tcc_body,
        grid=(25,),
        in_specs=[
            pl.BlockSpec((400, D), lambda i: (i, 0)),
            pl.BlockSpec((400, D), lambda i: (i, 0)),
            pl.BlockSpec((400, 1), lambda i: (i, 0)),
            pl.BlockSpec((400, 1), lambda i: (i, 0)),
        ],
        out_specs=pl.BlockSpec((400, D), lambda i: (i, 0)),
        out_shape=jax.ShapeDtypeStruct((N, D), _f32),
    )(h0, h1, d0, d1)


def kernel(feature, edge_index, edge_type, fc_weight, attn_weight):
    src = edge_index[0]
    dst = edge_index[1]
    et = edge_type

    w2 = fc_weight.transpose(1, 0, 2).reshape(D, R * D)
    w_s = attn_weight[:, :D, 0]
    w_d = attn_weight[:, D:, 0]
    eye = jnp.eye(R, dtype=_f32)
    wsel_s = (eye[:, None, :] * w_s[:, :, None]).reshape(R * D, R)
    wsel_d = (eye[:, None, :] * w_d[:, :, None]).reshape(R * D, R)
    wcat = jnp.concatenate([wsel_s, wsel_d], axis=1)

    zh, a_tab = _tc0(feature, w2, wcat)

    a_s_flat = a_tab[:, :R].reshape(-1)
    a_d_flat = jnp.pad(a_tab[:, R:].reshape(-1), (0, NP * R - N * R))

    is_ = src * R + et
    id_ = dst * R + et
    pad = EP - E
    is2d = jnp.pad(is_, (0, pad)).reshape(EP // 128, 128)
    id2d = jnp.pad(id_, (0, pad), constant_values=N * R).reshape(EP // 128, 128)

    e2d, m_part = _sc_a(is2d, id2d, a_s_flat, a_d_flat)

    zh_t = zh.reshape(N * R, D)
    hbar, den = _sc_b(is2d, id2d, e2d, m_part, zh_t)

    h = _tcc(hbar[0], hbar[1],
             den[0].reshape(NP, 1), den[1].reshape(NP, 1))
    return h


# speedup vs baseline: 20.1264x; 20.1264x over previous
import functools

import jax
import jax.numpy as jnp
from jax import lax
from jax.experimental import pallas as pl
from jax.experimental.pallas import tpu as pltpu
from jax.experimental.pallas import tpu_sc as plsc

N = 10000
E = 320000
D = 128
R = 8

NC = 2
NS = 16
L = 16
NW = NC * NS

NP = 10240
EP = 327680
EPT = EP // NW
MC = 1024
SUB = 128
CW = NP // NS

_f32 = jnp.float32


def _tc0_body(f_ref, w2_ref, wcat_ref, zh_ref, a_ref):
    zh = jnp.dot(f_ref[...], w2_ref[...], preferred_element_type=_f32)
    zh_ref[...] = zh
    a_ref[...] = jnp.dot(zh, wcat_ref[...], preferred_element_type=_f32)


def _tc0(feature, w2, wcat):
    return pl.pallas_call(
        _tc0_body,
        grid=(25,),
        in_specs=[
            pl.BlockSpec((400, D), lambda i: (i, 0)),
            pl.BlockSpec((D, R * D), lambda i: (0, 0)),
            pl.BlockSpec((R * D, 2 * R), lambda i: (0, 0)),
        ],
        out_specs=[
            pl.BlockSpec((400, R * D), lambda i: (i, 0)),
            pl.BlockSpec((400, 2 * R), lambda i: (i, 0)),
        ],
        out_shape=[
            jax.ShapeDtypeStruct((N, R * D), _f32),
            jax.ShapeDtypeStruct((N, 2 * R), _f32),
        ],
    )(feature, w2, wcat)


def _sc_a_body(is_hbm, id_hbm, as_hbm, ad_hbm,
               e_hbm, mpart_hbm,
               as_t, m_t, is_buf, id_buf, ad_buf, e_buf,
               red0, red1, stage, sem):
    cid = lax.axis_index("c")
    sid = lax.axis_index("s")
    wid = cid * NS + sid
    base_row = wid * (EPT // 128)

    pltpu.sync_copy(as_hbm, as_t)

    @pl.loop(0, NP // L)
    def _zero(v):
        m_t[pl.ds(v * L, L)] = jnp.zeros((L,), _f32)

    @pl.loop(0, EPT // MC)
    def _chunk(g):
        row = base_row + g * (MC // 128)
        pltpu.sync_copy(is_hbm.at[pl.ds(row, MC // 128)], is_buf)
        pltpu.sync_copy(id_hbm.at[pl.ds(row, MC // 128)], id_buf)

        @pl.loop(0, MC // 128)
        def _sub(j):
            pltpu.async_copy(ad_hbm.at[id_buf.at[j]], ad_buf.at[j], sem).wait()

            @pl.loop(0, 128 // L)
            def _vec(k):
                sl = pl.ds(k * L, L)
                is16 = is_buf[j, sl]
                id16 = id_buf[j, sl]
                as16 = plsc.load_gather(as_t, [is16])
                e16 = jnp.maximum(as16 + ad_buf[j, sl], 0.0)
                e_buf[j, sl] = e16
                d16 = lax.shift_right_logical(id16, 3)
                cur = plsc.load_gather(m_t, [d16])
                upd = jnp.maximum(cur, e16)
                plsc.store_scatter(m_t, [d16], upd)
                chk = plsc.load_gather(m_t, [d16])

                @pl.when(jnp.any(chk < e16))
                def _dup_fallback():
                    def _retry(_, pending):
                        cur2 = plsc.load_gather(m_t, [d16])
                        new2 = jnp.maximum(cur2, e16)
                        plsc.store_scatter(m_t, [d16], new2, mask=pending)
                        chk2 = plsc.load_gather(m_t, [d16])
                        return chk2 < e16

                    lax.fori_loop(0, L - 1, _retry, chk < e16)

        pltpu.sync_copy(e_buf, e_hbm.at[pl.ds(row, MC // 128)])

    pltpu.sync_copy(m_t, stage.at[sid])
    plsc.subcore_barrier()
    pltpu.sync_copy(stage.at[0, pl.ds(sid * CW, CW)], red0)
    for r in range(1, NS):
        pltpu.sync_copy(stage.at[r, pl.ds(sid * CW, CW)], red1)

        @pl.loop(0, CW // L)
        def _red(v):
            sl = pl.ds(v * L, L)
            red0[sl] = jnp.maximum(red0[sl], red1[sl])

    pltpu.sync_copy(red0, mpart_hbm.at[cid, pl.ds(sid * CW, CW)])


def _sc_a(is2d, id2d, a_s_flat, a_d_flat):
    mesh = plsc.VectorSubcoreMesh(core_axis_name="c", subcore_axis_name="s",
                                  num_cores=NC, num_subcores=NS)
    kern = pl.kernel(
        _sc_a_body,
        out_type=[
            jax.ShapeDtypeStruct((EP // 128, 128), _f32),
            jax.ShapeDtypeStruct((NC, NP), _f32),
        ],
        mesh=mesh,
        compiler_params=pltpu.CompilerParams(needs_layout_passes=False),
        scratch_types=[
            pltpu.VMEM((N * R,), _f32),
            pltpu.VMEM((NP,), _f32),
            pltpu.VMEM((MC // 128, 128), jnp.int32),
            pltpu.VMEM((MC // 128, 128), jnp.int32),
            pltpu.VMEM((MC // 128, 128), _f32),
            pltpu.VMEM((MC // 128, 128), _f32),
            pltpu.VMEM((CW,), _f32),
            pltpu.VMEM((CW,), _f32),
            pltpu.VMEM_SHARED((NS, NP), _f32),
            pltpu.SemaphoreType.DMA,
        ],
    )
    return kern(is2d, id2d, a_s_flat, a_d_flat)


def _sc_b_body(is_hbm, id_hbm, e_hbm, mpart_hbm, zh_hbm,
               hbar_hbm, den_hbm,
               m_t, mtmp, is_buf, id_buf, dst_buf, e_buf, ex_buf,
               rows, zbuf, zden, sph, spd, sem):
    cid = lax.axis_index("c")
    sid = lax.axis_index("s")
    wid = cid * NS + sid
    base_row = wid * (EPT // 128)

    pltpu.sync_copy(mpart_hbm.at[0], m_t)
    pltpu.sync_copy(mpart_hbm.at[1], mtmp)

    @pl.loop(0, NP // L)
    def _mmax(v):
        sl = pl.ds(v * L, L)
        m_t[sl] = jnp.maximum(m_t[sl], mtmp[sl])

    @pl.loop(0, L)
    def _zb(i):
        @pl.loop(0, D // L)
        def _zbi(k):
            zbuf[i, pl.ds(k * L, L)] = jnp.zeros((L,), _f32)

    @pl.loop(0, CW // L)
    def _zd(v):
        zden[pl.ds(v * L, L)] = jnp.zeros((L,), _f32)

    @pl.loop(0, CW // L)
    def _zh_rows(c):
        pltpu.sync_copy(zbuf, sph.at[pl.ds(sid * CW + c * L, L)])

    pltpu.sync_copy(zden, spd.at[pl.ds(sid * CW, CW)])
    plsc.subcore_barrier()

    @pl.loop(0, EPT // MC)
    def _chunk(g):
        row = base_row + g * (MC // 128)
        pltpu.sync_copy(is_hbm.at[pl.ds(row, MC // 128)], is_buf)
        pltpu.sync_copy(id_hbm.at[pl.ds(row, MC // 128)], id_buf)
        pltpu.sync_copy(e_hbm.at[pl.ds(row, MC // 128)], e_buf)

        @pl.loop(0, MC // SUB)
        def _sub(j):
            @pl.loop(0, SUB // L)
            def _vec(k):
                sl = pl.ds(k * L, L)
                d16 = lax.shift_right_logical(id_buf[j, sl], 3)
                dst_buf[j, sl] = d16
                mv = plsc.load_gather(m_t, [d16])
                ex_buf[j, sl] = jnp.exp(e_buf[j, sl] - mv)

            pltpu.sync_copy(ex_buf.at[j], spd.at[dst_buf.at[j]], add=True)
            pltpu.async_copy(zh_hbm.at[is_buf.at[j]], rows, sem).wait()

            @pl.loop(0, SUB)
            def _scale(r):
                ridx = jnp.full((L,), r, jnp.int32)
                exv = plsc.load_gather(ex_buf.at[j], [ridx])

                @pl.loop(0, D // L)
                def _sc(c):
                    sl = pl.ds(c * L, L)
                    rows[r, sl] = rows[r, sl] * exv

            pltpu.sync_copy(rows, sph.at[dst_buf.at[j]], add=True)

    plsc.subcore_barrier()
    pltpu.sync_copy(sph.at[pl.ds(sid * CW, CW)],
                    hbar_hbm.at[cid, pl.ds(sid * CW, CW)])
    pltpu.sync_copy(spd.at[pl.ds(sid * CW, CW)],
                    den_hbm.at[cid, pl.ds(sid * CW, CW)])


def _sc_b(is2d, id2d, e2d, m_part, zh_t):
    mesh = plsc.VectorSubcoreMesh(core_axis_name="c", subcore_axis_name="s",
                                  num_cores=NC, num_subcores=NS)
    kern = pl.kernel(
        _sc_b_body,
        out_type=[
            jax.ShapeDtypeStruct((NC, NP, D), _f32),
            jax.ShapeDtypeStruct((NC, NP), _f32),
        ],
        mesh=mesh,
        compiler_params=pltpu.CompilerParams(needs_layout_passes=False),
        scratch_types=[
            pltpu.VMEM((NP,), _f32),
            pltpu.VMEM((NP,), _f32),
            pltpu.VMEM((MC // 128, 128), jnp.int32),
            pltpu.VMEM((MC // 128, 128), jnp.int32),
            pltpu.VMEM((MC // 128, 128), jnp.int32),
            pltpu.VMEM((MC // 128, 128), _f32),
            pltpu.VMEM((MC // 128, 128), _f32),
            pltpu.VMEM((SUB, D), _f32),
            pltpu.VMEM((L, D), _f32),
            pltpu.VMEM((CW,), _f32),
            pltpu.VMEM_SHARED((NP, D), _f32),
            pltpu.VMEM_SHARED((NP,), _f32),
            pltpu.SemaphoreType.DMA,
        ],
    )
    return kern(is2d, id2d, e2d, m_part, zh_t)


def _tcc_body(h0_ref, h1_ref, d0_ref, d1_ref, out_ref):
    den = d0_ref[...] + d1_ref[...]
    den = jnp.where(den == 0.0, 1.0, den)
    out_ref[...] = (h0_ref[...] + h1_ref[...]) / den


def _tcc(h0, h1, d0, d1):
    return pl.pallas_call(
        _tcc_body,
        grid=(25,),
        in_specs=[
            pl.BlockSpec((400, D), lambda i: (i, 0)),
            pl.BlockSpec((400, D), lambda i: (i, 0)),
            pl.BlockSpec((400, 1), lambda i: (i, 0)),
            pl.BlockSpec((400, 1), lambda i: (i, 0)),
        ],
        out_specs=pl.BlockSpec((400, D), lambda i: (i, 0)),
        out_shape=jax.ShapeDtypeStruct((N, D), _f32),
    )(h0, h1, d0, d1)


def kernel(feature, edge_index, edge_type, fc_weight, attn_weight):
    src = edge_index[0]
    dst = edge_index[1]
    et = edge_type

    w2 = fc_weight.transpose(1, 0, 2).reshape(D, R * D)
    w_s = attn_weight[:, :D, 0]
    w_d = attn_weight[:, D:, 0]
    eye = jnp.eye(R, dtype=_f32)
    wsel_s = (eye[:, None, :] * w_s[:, :, None]).reshape(R * D, R)
    wsel_d = (eye[:, None, :] * w_d[:, :, None]).reshape(R * D, R)
    wcat = jnp.concatenate([wsel_s, wsel_d], axis=1)

    zh, a_tab = _tc0(feature, w2, wcat)

    a_s_flat = a_tab[:, :R].reshape(-1)
    a_d_flat = jnp.pad(a_tab[:, R:].reshape(-1), (0, NP * R - N * R))

    is_ = src * R + et
    id_ = dst * R + et
    pad = EP - E
    is2d = jnp.pad(is_, (0, pad)).reshape(EP // 128, 128)
    id2d = jnp.pad(id_, (0, pad), constant_values=N * R).reshape(EP // 128, 128)

    e2d, m_part = _sc_a(is2d, id2d, a_s_flat, a_d_flat)

    zh_t = zh.reshape(N * R, D)
    hbar, den = _sc_b(is2d, id2d, e2d, m_part, zh_t)

    h = _tcc(hbar[0], hbar[1],
             den[0].reshape(NP, 1), den[1].reshape(NP, 1))
    return h


# speedup vs baseline: 23.9902x; 1.1920x over previous
import functools

import jax
import jax.numpy as jnp
from jax import lax
from jax.experimental import pallas as pl
from jax.experimental.pallas import tpu as pltpu
from jax.experimental.pallas import tpu_sc as plsc

N = 10000
E = 320000
D = 128
R = 8

NC = 2
NS = 16
L = 16
NW = NC * NS

NP = 10240
EP = 327680
EPT = EP // NW
MC = 1024
SUB = 128
CW = NP // NS

_f32 = jnp.float32


def _tc0_body(f_ref, w2_ref, wcat_ref, zh_ref, a_ref):
    zh = jnp.dot(f_ref[...], w2_ref[...], preferred_element_type=_f32)
    zh_ref[...] = zh
    a_ref[...] = jnp.dot(zh, wcat_ref[...], preferred_element_type=_f32)


def _tc0(feature, w2, wcat):
    return pl.pallas_call(
        _tc0_body,
        grid=(25,),
        in_specs=[
            pl.BlockSpec((400, D), lambda i: (i, 0)),
            pl.BlockSpec((D, R * D), lambda i: (0, 0)),
            pl.BlockSpec((R * D, 2 * R), lambda i: (0, 0)),
        ],
        out_specs=[
            pl.BlockSpec((400, R * D), lambda i: (i, 0)),
            pl.BlockSpec((400, 2 * R), lambda i: (i, 0)),
        ],
        out_shape=[
            jax.ShapeDtypeStruct((N, R * D), _f32),
            jax.ShapeDtypeStruct((N, 2 * R), _f32),
        ],
    )(feature, w2, wcat)


def _sc_a_body(is_hbm, id_hbm, as_hbm, ad_hbm,
               e_hbm, mpart_hbm,
               as_t, m_t, is_buf, id_buf, ad_buf, e_buf,
               red0, red1, stage, sem):
    cid = lax.axis_index("c")
    sid = lax.axis_index("s")
    wid = cid * NS + sid
    base_row = wid * (EPT // 128)

    pltpu.sync_copy(as_hbm, as_t)

    @pl.loop(0, NP // L)
    def _zero(v):
        m_t[pl.ds(v * L, L)] = jnp.zeros((L,), _f32)

    @pl.loop(0, EPT // MC)
    def _chunk(g):
        row = base_row + g * (MC // 128)
        pltpu.sync_copy(is_hbm.at[pl.ds(row, MC // 128)], is_buf)
        pltpu.sync_copy(id_hbm.at[pl.ds(row, MC // 128)], id_buf)

        descs = [pltpu.async_copy(ad_hbm.at[id_buf.at[j]], ad_buf.at[j], sem)
                 for j in range(MC // 128)]
        for j in range(MC // 128):
            descs[j].wait()

            @pl.loop(0, 128 // L)
            def _vec(k):
                sl = pl.ds(k * L, L)
                is16 = is_buf[j, sl]
                id16 = id_buf[j, sl]
                as16 = plsc.load_gather(as_t, [is16])
                e16 = jnp.maximum(as16 + ad_buf[j, sl], 0.0)
                e_buf[j, sl] = e16
                d16 = lax.shift_right_logical(id16, 3)
                cur = plsc.load_gather(m_t, [d16])
                upd = jnp.maximum(cur, e16)
                plsc.store_scatter(m_t, [d16], upd)
                chk = plsc.load_gather(m_t, [d16])

                @pl.when(jnp.any(chk < e16))
                def _dup_fallback():
                    def _retry(_, pending):
                        cur2 = plsc.load_gather(m_t, [d16])
                        new2 = jnp.maximum(cur2, e16)
                        plsc.store_scatter(m_t, [d16], new2, mask=pending)
                        chk2 = plsc.load_gather(m_t, [d16])
                        return chk2 < e16

                    lax.fori_loop(0, L - 1, _retry, chk < e16)

        pltpu.sync_copy(e_buf, e_hbm.at[pl.ds(row, MC // 128)])

    pltpu.sync_copy(m_t, stage.at[sid])
    plsc.subcore_barrier()
    pltpu.sync_copy(stage.at[0, pl.ds(sid * CW, CW)], red0)
    for r in range(1, NS):
        pltpu.sync_copy(stage.at[r, pl.ds(sid * CW, CW)], red1)

        @pl.loop(0, CW // L)
        def _red(v):
            sl = pl.ds(v * L, L)
            red0[sl] = jnp.maximum(red0[sl], red1[sl])

    pltpu.sync_copy(red0, mpart_hbm.at[cid, pl.ds(sid * CW, CW)])


def _sc_a(is2d, id2d, a_s_flat, a_d_flat):
    mesh = plsc.VectorSubcoreMesh(core_axis_name="c", subcore_axis_name="s",
                                  num_cores=NC, num_subcores=NS)
    kern = pl.kernel(
        _sc_a_body,
        out_type=[
            jax.ShapeDtypeStruct((EP // 128, 128), _f32),
            jax.ShapeDtypeStruct((NC, NP), _f32),
        ],
        mesh=mesh,
        compiler_params=pltpu.CompilerParams(needs_layout_passes=False),
        scratch_types=[
            pltpu.VMEM((N * R,), _f32),
            pltpu.VMEM((NP,), _f32),
            pltpu.VMEM((MC // 128, 128), jnp.int32),
            pltpu.VMEM((MC // 128, 128), jnp.int32),
            pltpu.VMEM((MC // 128, 128), _f32),
            pltpu.VMEM((MC // 128, 128), _f32),
            pltpu.VMEM((CW,), _f32),
            pltpu.VMEM((CW,), _f32),
            pltpu.VMEM_SHARED((NS, NP), _f32),
            pltpu.SemaphoreType.DMA,
        ],
    )
    return kern(is2d, id2d, a_s_flat, a_d_flat)


SUBB = 64
NSUB = MC // SUBB


def _sc_b_body(is_hbm, id_hbm, e_hbm, m_hbm, zh_hbm,
               hbar_hbm, den_hbm,
               m_t, is_buf, id_buf, dst_buf, e_buf, ex_buf,
               rows, zden, sph, spd, gsems, ssems, dsem):
    cid = lax.axis_index("c")
    sid = lax.axis_index("s")
    wid = cid * NS + sid
    base_row = wid * (EPT // SUBB)

    pltpu.sync_copy(m_hbm, m_t)

    @pl.loop(0, SUBB)
    def _zb(i):
        @pl.loop(0, D // L)
        def _zbi(k):
            rows[0, i, pl.ds(k * L, L)] = jnp.zeros((L,), _f32)

    @pl.loop(0, CW // L)
    def _zd(v):
        zden[pl.ds(v * L, L)] = jnp.zeros((L,), _f32)

    @pl.loop(0, CW // SUBB)
    def _zh_rows(c):
        pltpu.sync_copy(rows.at[0], sph.at[pl.ds(sid * CW + c * SUBB, SUBB)])

    pltpu.sync_copy(zden, spd.at[pl.ds(sid * CW, CW)])
    plsc.subcore_barrier()

    @pl.loop(0, EPT // MC)
    def _chunk(g):
        row = base_row + g * NSUB
        pltpu.sync_copy(is_hbm.at[pl.ds(row, NSUB)], is_buf)
        pltpu.sync_copy(id_hbm.at[pl.ds(row, NSUB)], id_buf)
        pltpu.sync_copy(e_hbm.at[pl.ds(row, NSUB)], e_buf)

        @pl.loop(0, MC // L)
        def _vec(v):
            j = v // (SUBB // L)
            sl = pl.ds((v % (SUBB // L)) * L, L)
            d16 = lax.shift_right_logical(id_buf[j, sl], 3)
            dst_buf[j, sl] = d16
            mv = plsc.load_gather(m_t, [d16])
            ex_buf[j, sl] = jnp.exp(e_buf[j, sl] - mv)

        gd = [None] * NSUB
        sd = [None] * NSUB
        dd = [None] * NSUB
        gd[0] = pltpu.async_copy(zh_hbm.at[is_buf.at[0]], rows.at[0], gsems[0])
        for j in range(NSUB):
            b = j % 2
            if j + 1 < NSUB:
                if j >= 1:
                    sd[j - 1].wait()
                gd[j + 1] = pltpu.async_copy(
                    zh_hbm.at[is_buf.at[j + 1]], rows.at[(j + 1) % 2],
                    gsems[(j + 1) % 2])
            gd[j].wait()
            dd[j] = pltpu.async_copy(ex_buf.at[j], spd.at[dst_buf.at[j]],
                                     dsem, add=True)

            @pl.loop(0, SUBB)
            def _scale(r, j=j, b=b):
                ridx = jnp.full((L,), r, jnp.int32)
                exv = plsc.load_gather(ex_buf.at[j], [ridx])

                @pl.loop(0, D // L)
                def _sc(c):
                    sl = pl.ds(c * L, L)
                    rows[b, r, sl] = rows[b, r, sl] * exv

            sd[j] = pltpu.async_copy(rows.at[b], sph.at[dst_buf.at[j]],
                                     ssems[b], add=True)
        sd[NSUB - 2].wait()
        sd[NSUB - 1].wait()
        for j in range(NSUB):
            dd[j].wait()

    plsc.subcore_barrier()
    pltpu.sync_copy(sph.at[pl.ds(sid * CW, CW)],
                    hbar_hbm.at[cid, pl.ds(sid * CW, CW)])
    pltpu.sync_copy(spd.at[pl.ds(sid * CW, CW)],
                    den_hbm.at[cid, pl.ds(sid * CW, CW)])


def _sc_b(is2d, id2d, e2d, m_comb, zh_t):
    mesh = plsc.VectorSubcoreMesh(core_axis_name="c", subcore_axis_name="s",
                                  num_cores=NC, num_subcores=NS)
    kern = pl.kernel(
        _sc_b_body,
        out_type=[
            jax.ShapeDtypeStruct((NC, NP, D), _f32),
            jax.ShapeDtypeStruct((NC, NP), _f32),
        ],
        mesh=mesh,
        compiler_params=pltpu.CompilerParams(needs_layout_passes=False),
        scratch_types=[
            pltpu.VMEM((NP,), _f32),
            pltpu.VMEM((NSUB, SUBB), jnp.int32),
            pltpu.VMEM((NSUB, SUBB), jnp.int32),
            pltpu.VMEM((NSUB, SUBB), jnp.int32),
            pltpu.VMEM((NSUB, SUBB), _f32),
            pltpu.VMEM((NSUB, SUBB), _f32),
            pltpu.VMEM((2, SUBB, D), _f32),
            pltpu.VMEM((CW,), _f32),
            pltpu.VMEM_SHARED((NP, D), _f32),
            pltpu.VMEM_SHARED((NP,), _f32),
            [pltpu.SemaphoreType.DMA, pltpu.SemaphoreType.DMA],
            [pltpu.SemaphoreType.DMA, pltpu.SemaphoreType.DMA],
            pltpu.SemaphoreType.DMA,
        ],
    )
    return kern(is2d, id2d, e2d, m_comb, zh_t)


def _tcm_body(mp_ref, out_ref):
    out_ref[...] = jnp.max(mp_ref[...], axis=0, keepdims=True)


def _tcm(m_part):
    out = pl.pallas_call(
        _tcm_body,
        out_shape=jax.ShapeDtypeStruct((1, NP), _f32),
    )(m_part)
    return out.reshape(NP)


def _tcc_body(h0_ref, h1_ref, d0_ref, d1_ref, out_ref):
    den = d0_ref[...] + d1_ref[...]
    den = jnp.where(den == 0.0, 1.0, den)
    out_ref[...] = (h0_ref[...] + h1_ref[...]) / den


def _tcc(h0, h1, d0, d1):
    return pl.pallas_call(
        _tcc_body,
        grid=(25,),
        in_specs=[
            pl.BlockSpec((400, D), lambda i: (i, 0)),
            pl.BlockSpec((400, D), lambda i: (i, 0)),
            pl.BlockSpec((400, 1), lambda i: (i, 0)),
            pl.BlockSpec((400, 1), lambda i: (i, 0)),
        ],
        out_specs=pl.BlockSpec((400, D), lambda i: (i, 0)),
        out_shape=jax.ShapeDtypeStruct((N, D), _f32),
    )(h0, h1, d0, d1)


def kernel(feature, edge_index, edge_type, fc_weight, attn_weight):
    src = edge_index[0]
    dst = edge_index[1]
    et = edge_type

    w2 = fc_weight.transpose(1, 0, 2).reshape(D, R * D)
    w_s = attn_weight[:, :D, 0]
    w_d = attn_weight[:, D:, 0]
    eye = jnp.eye(R, dtype=_f32)
    wsel_s = (eye[:, None, :] * w_s[:, :, None]).reshape(R * D, R)
    wsel_d = (eye[:, None, :] * w_d[:, :, None]).reshape(R * D, R)
    wcat = jnp.concatenate([wsel_s, wsel_d], axis=1)

    zh, a_tab = _tc0(feature, w2, wcat)

    a_s_flat = a_tab[:, :R].reshape(-1)
    a_d_flat = jnp.pad(a_tab[:, R:].reshape(-1), (0, NP * R - N * R))

    is_ = src * R + et
    id_ = dst * R + et
    pad = EP - E
    is2d = jnp.pad(is_, (0, pad)).reshape(EP // 128, 128)
    id2d = jnp.pad(id_, (0, pad), constant_values=N * R).reshape(EP // 128, 128)

    e2d, m_part = _sc_a(is2d, id2d, a_s_flat, a_d_flat)
    m_comb = _tcm(m_part)

    zh_t = zh.reshape(N * R, D)
    hbar, den = _sc_b(is2d.reshape(EP // 64, 64), id2d.reshape(EP // 64, 64),
                      e2d.reshape(EP // 64, 64), m_comb, zh_t)

    h = _tcc(hbar[0], hbar[1],
             den[0].reshape(NP, 1), den[1].reshape(NP, 1))
    return h
